# baseline (speedup 1.0000x reference)
import jax
import jax.numpy as jnp
from jax import lax
from jax.experimental import pallas as pl
from jax.experimental.pallas import tpu as pltpu
from jax.experimental.pallas import tpu_sc as plsc

N = 100000
C = 128
NC, NS = 2, 16
NW = NC * NS
RPW = 3136
NP = NW * RPW
CH = 224
NCHUNK = RPW // CH
G = CH // 16


def _onehot_body(idx_hbm, zero_hbm, out_hbm, idx_v, buf0, buf1, sem0, sem1):
    wid = lax.axis_index("s") * NC + lax.axis_index("c")
    base = wid * RPW
    bufs = (buf0, buf1)
    sems = (sem0, sem1)

    fill0 = pltpu.async_copy(zero_hbm, buf0, sem0)
    fill1 = pltpu.async_copy(zero_hbm, buf1, sem1)
    pltpu.sync_copy(idx_hbm.at[pl.ds(base, RPW)], idx_v)

    rows0 = lax.iota(jnp.int32, 16) * C
    ones = jnp.ones((16,), jnp.int32)
    zeros = jnp.zeros((16,), jnp.int32)

    pending = [fill0, fill1]
    for k in range(NCHUNK):
        b = k % 2
        pending[b].wait()
        for g in range(G):
            rows = rows0 + (g * 16 * C)
            if k >= 2:
                old = idx_v[pl.ds((k - 2) * CH + g * 16, 16)]
                plsc.store_scatter(bufs[b], [rows + old], zeros)
            cols = idx_v[pl.ds(k * CH + g * 16, 16)]
            plsc.store_scatter(bufs[b], [rows + cols], ones)
        pending[b] = pltpu.async_copy(
            bufs[b], out_hbm.at[pl.ds((base + k * CH) * C, CH * C)], sems[b])
    pending[0].wait()
    pending[1].wait()


_onehot_sc = pl.kernel(
    _onehot_body,
    out_type=jax.ShapeDtypeStruct((NP * C,), jnp.int32),
    mesh=plsc.VectorSubcoreMesh(core_axis_name="c", subcore_axis_name="s"),
    compiler_params=pltpu.CompilerParams(needs_layout_passes=False),
    scratch_types=[
        pltpu.VMEM((RPW,), jnp.int32),
        pltpu.VMEM((CH * C,), jnp.int32),
        pltpu.VMEM((CH * C,), jnp.int32),
        pltpu.SemaphoreType.DMA,
        pltpu.SemaphoreType.DMA,
    ],
)


def kernel(x, weight):
    del x
    idx = jnp.pad(weight, (0, NP - N))
    zero_chunk = jnp.zeros((CH * C,), jnp.int32)
    out = _onehot_sc(idx, zero_chunk)
    return out.reshape(NP, C)[:N]


# speedup vs baseline: 1.5873x; 1.5873x over previous
import jax
import jax.numpy as jnp
from jax import lax
from jax.experimental import pallas as pl
from jax.experimental.pallas import tpu as pltpu
from jax.experimental.pallas import tpu_sc as plsc

N = 100000
C = 128
NC, NS = 2, 16
NW = NC * NS
R = N // NW
CH = 224
NCHUNK = -(-R // CH)
LASTCH = R - (NCHUNK - 1) * CH
G = CH // 16
LASTG = -(-LASTCH // 16)
TAIL = LASTCH - (LASTG - 1) * 16
IDXV = (NCHUNK - 1) * CH + LASTG * 16 + 8
NIDX = (NW - 1) * R - ((NW - 1) * R) % 8 + IDXV


def _onehot_body(idx_hbm, zero_hbm, out_hbm, idx_v, buf0, buf1, sem0, sem1):
    wid = lax.axis_index("s") * NC + lax.axis_index("c")
    base = wid * R
    abase = pl.multiple_of(base - base % 8, 8)
    off = base % 8
    bufs = (buf0, buf1)
    sems = (sem0, sem1)

    fill0 = pltpu.async_copy(zero_hbm, buf0, sem0)
    fill1 = pltpu.async_copy(zero_hbm, buf1, sem1)
    pltpu.sync_copy(idx_hbm.at[pl.ds(abase, IDXV)], idx_v)

    rows0 = lax.iota(jnp.int32, 16) * C
    ones = jnp.ones((16,), jnp.int32)
    zeros = jnp.zeros((16,), jnp.int32)
    tailmask = lax.iota(jnp.int32, 16) < TAIL

    pending = [fill0, fill1]
    for k in range(NCHUNK):
        b = k % 2
        last = k == NCHUNK - 1
        pending[b].wait()
        for g in range(G):
            rows = rows0 + (g * 16 * C)
            if k >= 2:
                old = idx_v[pl.ds(off + (k - 2) * CH + g * 16, 16)]
                plsc.store_scatter(bufs[b], [rows + old], zeros)
            cols = idx_v[pl.ds(off + k * CH + g * 16, 16)]
            mask = tailmask if (last and g == LASTG - 1) else None
            plsc.store_scatter(bufs[b], [rows + cols], ones, mask=mask)
        nrows = LASTCH if last else CH
        pending[b] = pltpu.async_copy(
            bufs[b].at[pl.ds(0, nrows * C)],
            out_hbm.at[pl.ds((base + k * CH) * C, nrows * C)],
            sems[b])
    pending[0].wait()
    pending[1].wait()


_onehot_sc = pl.kernel(
    _onehot_body,
    out_type=jax.ShapeDtypeStruct((N * C,), jnp.int32),
    mesh=plsc.VectorSubcoreMesh(core_axis_name="c", subcore_axis_name="s"),
    compiler_params=pltpu.CompilerParams(needs_layout_passes=False),
    scratch_types=[
        pltpu.VMEM((IDXV,), jnp.int32),
        pltpu.VMEM((CH * C,), jnp.int32),
        pltpu.VMEM((CH * C,), jnp.int32),
        pltpu.SemaphoreType.DMA,
        pltpu.SemaphoreType.DMA,
    ],
)


def kernel(x, weight):
    del x
    idx = jnp.pad(weight, (0, NIDX - N))
    zero_chunk = jnp.zeros((CH * C,), jnp.int32)
    out = _onehot_sc(idx, zero_chunk)
    return out.reshape(N, C)
